# baseline (speedup 1.0000x reference)
import functools

import jax
import jax.numpy as jnp
from jax.experimental import pallas as pl


def _router_body(x_ref, w_ref, b_ref, topi_ref, topw_ref, mu_ref, *, n_tokens):
    logits = (
        jnp.dot(x_ref[...], w_ref[...], preferred_element_type=jnp.float32)
        + b_ref[...]
    )
    e_dim = logits.shape[-1]
    m = jnp.max(logits, axis=-1, keepdims=True)
    ex = jnp.exp(logits - m)
    s = jnp.sum(ex, axis=-1, keepdims=True)
    gates = ex / s

    iota = jax.lax.broadcasted_iota(jnp.int32, gates.shape, 1)
    m1 = jnp.max(gates, axis=-1, keepdims=True)
    i1 = jnp.min(
        jnp.where(gates == m1, iota, e_dim), axis=-1, keepdims=True
    )
    masked = jnp.where(iota == i1, -jnp.inf, gates)
    m2 = jnp.max(masked, axis=-1, keepdims=True)
    i2 = jnp.min(
        jnp.where(masked == m2, iota, e_dim), axis=-1, keepdims=True
    )

    topw_ref[...] = jnp.concatenate([m1, m2], axis=1)
    topi_ref[...] = jnp.concatenate([i1, i2], axis=1)

    part = jnp.sum(gates, axis=0, keepdims=True) * (1.0 / n_tokens)

    @pl.when(pl.program_id(0) == 0)
    def _init():
        mu_ref[...] = jnp.zeros_like(mu_ref)

    mu_ref[...] += part


def kernel(x, W, b):
    t, d = x.shape
    e = W.shape[1]
    bt = 512
    grid = (t // bt,)

    b2 = b.reshape(1, e)

    topi, topw, mu = pl.pallas_call(
        functools.partial(_router_body, n_tokens=t),
        grid=grid,
        in_specs=[
            pl.BlockSpec((bt, d), lambda i: (i, 0)),
            pl.BlockSpec((d, e), lambda i: (0, 0)),
            pl.BlockSpec((1, e), lambda i: (0, 0)),
        ],
        out_specs=[
            pl.BlockSpec((bt, 2), lambda i: (i, 0)),
            pl.BlockSpec((bt, 2), lambda i: (i, 0)),
            pl.BlockSpec((1, e), lambda i: (0, 0)),
        ],
        out_shape=[
            jax.ShapeDtypeStruct((t, 2), jnp.int32),
            jax.ShapeDtypeStruct((t, 2), jnp.float32),
            jax.ShapeDtypeStruct((1, e), jnp.float32),
        ],
    )(x, W, b2)

    return (topi, topw, mu.reshape(e))


# speedup vs baseline: 1.1640x; 1.1640x over previous
import functools

import jax
import jax.numpy as jnp
from jax.experimental import pallas as pl


def _router_body(x_ref, w_ref, b_ref, topi_ref, topw_ref, mu_ref, *, n_tokens):
    logits = (
        jnp.dot(x_ref[...], w_ref[...], preferred_element_type=jnp.float32)
        + b_ref[...]
    )
    e_dim = logits.shape[-1]
    m = jnp.max(logits, axis=-1, keepdims=True)
    ex = jnp.exp(logits - m)
    s = jnp.sum(ex, axis=-1, keepdims=True)
    gates = ex / s

    iota = jax.lax.broadcasted_iota(jnp.int32, gates.shape, 1)
    m1 = jnp.max(gates, axis=-1, keepdims=True)
    i1 = jnp.min(
        jnp.where(gates == m1, iota, e_dim), axis=-1, keepdims=True
    )
    masked = jnp.where(iota == i1, -jnp.inf, gates)
    m2 = jnp.max(masked, axis=-1, keepdims=True)
    i2 = jnp.min(
        jnp.where(masked == m2, iota, e_dim), axis=-1, keepdims=True
    )

    topw_ref[...] = jnp.concatenate([m1, m2], axis=1)
    topi_ref[...] = jnp.concatenate([i1, i2], axis=1)

    part = jnp.sum(gates, axis=0, keepdims=True) * (1.0 / n_tokens)

    @pl.when(pl.program_id(0) == 0)
    def _init():
        mu_ref[...] = jnp.zeros_like(mu_ref)

    mu_ref[...] += part


def kernel(x, W, b):
    t, d = x.shape
    e = W.shape[1]
    bt = 2048
    grid = (t // bt,)

    b2 = b.reshape(1, e)

    topi, topw, mu = pl.pallas_call(
        functools.partial(_router_body, n_tokens=t),
        grid=grid,
        in_specs=[
            pl.BlockSpec((bt, d), lambda i: (i, 0)),
            pl.BlockSpec((d, e), lambda i: (0, 0)),
            pl.BlockSpec((1, e), lambda i: (0, 0)),
        ],
        out_specs=[
            pl.BlockSpec((bt, 2), lambda i: (i, 0)),
            pl.BlockSpec((bt, 2), lambda i: (i, 0)),
            pl.BlockSpec((1, e), lambda i: (0, 0)),
        ],
        out_shape=[
            jax.ShapeDtypeStruct((t, 2), jnp.int32),
            jax.ShapeDtypeStruct((t, 2), jnp.float32),
            jax.ShapeDtypeStruct((1, e), jnp.float32),
        ],
    )(x, W, b2)

    return (topi, topw, mu.reshape(e))


# speedup vs baseline: 1.1836x; 1.0168x over previous
import functools

import jax
import jax.numpy as jnp
from jax.experimental import pallas as pl


def _router_body(x_ref, w_ref, b_ref, topi_ref, topw_ref, mu_ref, *, n_tokens):
    logits = (
        jnp.dot(x_ref[...], w_ref[...], preferred_element_type=jnp.float32)
        + b_ref[...]
    )
    e_dim = logits.shape[-1]
    iota = jax.lax.broadcasted_iota(jnp.int32, logits.shape, 1)

    m = jnp.max(logits, axis=-1, keepdims=True)
    i1 = jnp.min(
        jnp.where(logits == m, iota, e_dim), axis=-1, keepdims=True
    )
    masked = jnp.where(iota == i1, -jnp.inf, logits)
    m2 = jnp.max(masked, axis=-1, keepdims=True)
    i2 = jnp.min(
        jnp.where(masked == m2, iota, e_dim), axis=-1, keepdims=True
    )

    ex = jnp.exp(logits - m)
    s = jnp.sum(ex, axis=-1, keepdims=True)
    r = 1.0 / s
    topw_ref[...] = jnp.concatenate([r, jnp.exp(m2 - m) * r], axis=1)
    topi_ref[...] = jnp.concatenate([i1, i2], axis=1)

    part = jnp.sum(ex * r, axis=0, keepdims=True) * (1.0 / n_tokens)

    @pl.when(pl.program_id(0) == 0)
    def _init():
        mu_ref[...] = jnp.zeros_like(mu_ref)

    mu_ref[...] += part


def kernel(x, W, b):
    t, d = x.shape
    e = W.shape[1]
    bt = 2048
    grid = (t // bt,)

    b2 = b.reshape(1, e)

    topi, topw, mu = pl.pallas_call(
        functools.partial(_router_body, n_tokens=t),
        grid=grid,
        in_specs=[
            pl.BlockSpec((bt, d), lambda i: (i, 0)),
            pl.BlockSpec((d, e), lambda i: (0, 0)),
            pl.BlockSpec((1, e), lambda i: (0, 0)),
        ],
        out_specs=[
            pl.BlockSpec((bt, 2), lambda i: (i, 0)),
            pl.BlockSpec((bt, 2), lambda i: (i, 0)),
            pl.BlockSpec((1, e), lambda i: (0, 0)),
        ],
        out_shape=[
            jax.ShapeDtypeStruct((t, 2), jnp.int32),
            jax.ShapeDtypeStruct((t, 2), jnp.float32),
            jax.ShapeDtypeStruct((1, e), jnp.float32),
        ],
    )(x, W, b2)

    return (topi, topw, mu.reshape(e))


# speedup vs baseline: 1.2540x; 1.0595x over previous
import functools

import jax
import jax.numpy as jnp
from jax.experimental import pallas as pl


def _router_body(x_ref, w_ref, b_ref, topi_ref, topw_ref, mu_ref, *, n_tokens):
    logits = (
        jnp.dot(x_ref[...], w_ref[...], preferred_element_type=jnp.float32)
        + b_ref[...]
    )
    lt = logits.T
    e_dim = lt.shape[0]
    iota = jax.lax.broadcasted_iota(jnp.int32, lt.shape, 0)

    m = jnp.max(lt, axis=0, keepdims=True)
    i1 = jnp.min(jnp.where(lt == m, iota, e_dim), axis=0, keepdims=True)
    masked = jnp.where(iota == i1, -jnp.inf, lt)
    m2 = jnp.max(masked, axis=0, keepdims=True)
    i2 = jnp.min(jnp.where(masked == m2, iota, e_dim), axis=0, keepdims=True)

    ex = jnp.exp(lt - m)
    s = jnp.sum(ex, axis=0, keepdims=True)
    r = 1.0 / s
    topw_ref[...] = jnp.concatenate([r, jnp.exp(m2 - m) * r], axis=0).T
    topi_ref[...] = jnp.concatenate([i1, i2], axis=0).T

    part = jnp.sum(ex * r, axis=1, keepdims=True) * (1.0 / n_tokens)

    @pl.when(pl.program_id(0) == 0)
    def _init():
        mu_ref[...] = jnp.zeros_like(mu_ref)

    mu_ref[...] += part


def kernel(x, W, b):
    t, d = x.shape
    e = W.shape[1]
    bt = 1024
    grid = (t // bt,)

    b2 = b.reshape(1, e)

    topi, topw, mu = pl.pallas_call(
        functools.partial(_router_body, n_tokens=t),
        grid=grid,
        in_specs=[
            pl.BlockSpec((bt, d), lambda i: (i, 0)),
            pl.BlockSpec((d, e), lambda i: (0, 0)),
            pl.BlockSpec((1, e), lambda i: (0, 0)),
        ],
        out_specs=[
            pl.BlockSpec((bt, 2), lambda i: (i, 0)),
            pl.BlockSpec((bt, 2), lambda i: (i, 0)),
            pl.BlockSpec((e, 1), lambda i: (0, 0)),
        ],
        out_shape=[
            jax.ShapeDtypeStruct((t, 2), jnp.int32),
            jax.ShapeDtypeStruct((t, 2), jnp.float32),
            jax.ShapeDtypeStruct((e, 1), jnp.float32),
        ],
    )(x, W, b2)

    return (topi, topw, mu.reshape(e))
